# baseline (speedup 1.0000x reference)
import functools

import jax
import jax.numpy as jnp
from jax import lax
from jax.experimental import pallas as pl
from jax.experimental.pallas import tpu as pltpu
from jax.experimental.pallas import tpu_sc as plsc

_VB = 2048
_IDX_CHUNK = 128


def _gather_mean_sc(idx_flat, emb, batch, win):
    info = plsc.get_sparse_core_info()
    nc, ns = info.num_cores, info.num_subcores
    nw = nc * ns
    edim = emb.shape[1]
    b_per_w = batch // nw
    rows_per_w = b_per_w * win
    n_chunks = rows_per_w // _IDX_CHUNK
    mesh = plsc.VectorSubcoreMesh(core_axis_name="c", subcore_axis_name="s")

    def body(idx_hbm, emb_hbm, out_hbm, idx_v, rows_v, acc_v, sem):
        wid = lax.axis_index("s") * nc + lax.axis_index("c")
        base = wid * rows_per_w
        pltpu.sync_copy(idx_hbm.at[pl.ds(base, rows_per_w)], idx_v)
        copies = []
        for j in range(n_chunks):
            copies.append(pltpu.async_copy(
                emb_hbm.at[idx_v.at[pl.ds(j * _IDX_CHUNK, _IDX_CHUNK)]],
                rows_v.at[pl.ds(j * _IDX_CHUNK, _IDX_CHUNK)],
                sem))
        for c in copies:
            c.wait()

        def pool_one(i, carry):
            acc = rows_v[i * win, :]
            for j in range(1, win):
                acc = acc + rows_v[i * win + j, :]
            acc_v[i, :] = acc * (1.0 / win)
            return carry

        lax.fori_loop(0, b_per_w, pool_one, 0)
        pltpu.sync_copy(acc_v, out_hbm.at[pl.ds(wid * b_per_w, b_per_w)])

    kfn = pl.kernel(
        body,
        mesh=mesh,
        compiler_params=pltpu.CompilerParams(use_tc_tiling_on_sc=False),
        out_type=jax.ShapeDtypeStruct((batch, edim), jnp.float32),
        scratch_types=[
            pltpu.VMEM((rows_per_w,), jnp.int32),
            pltpu.VMEM((rows_per_w, edim), jnp.float32),
            pltpu.VMEM((b_per_w, edim), jnp.float32),
            pltpu.SemaphoreType.DMA,
        ],
    )
    return kfn(idx_flat, emb)


def _lse_tc(v, W, b2, vocab):
    batch, k = v.shape
    nv = pl.cdiv(vocab, _VB)

    def body(v_ref, w_ref, b_ref, lse_ref, m_s, s_s):
        iv = pl.program_id(0)
        logits = lax.dot_general(v_ref[:], w_ref[:], (((1,), (1,)), ((), ())),
                                 preferred_element_type=jnp.float32) + b_ref[:]
        col = iv * _VB + lax.broadcasted_iota(jnp.int32, logits.shape, 1)
        logits = jnp.where(col < vocab, logits, -1e30)
        tmax = jnp.max(logits, axis=1, keepdims=True)

        @pl.when(iv == 0)
        def _():
            m_s[:] = tmax
            s_s[:] = jnp.sum(jnp.exp(logits - tmax), axis=1, keepdims=True)

        @pl.when(iv != 0)
        def _():
            m_old = m_s[:]
            m_new = jnp.maximum(m_old, tmax)
            s_s[:] = (s_s[:] * jnp.exp(m_old - m_new)
                      + jnp.sum(jnp.exp(logits - m_new), axis=1,
                                keepdims=True))
            m_s[:] = m_new

        @pl.when(iv == nv - 1)
        def _():
            lse_ref[:] = m_s[:] + jnp.log(s_s[:])

    return pl.pallas_call(
        body,
        grid=(nv,),
        in_specs=[
            pl.BlockSpec((batch, k), lambda i: (0, 0)),
            pl.BlockSpec((_VB, k), lambda i: (i, 0)),
            pl.BlockSpec((1, _VB), lambda i: (0, i)),
        ],
        out_specs=pl.BlockSpec((batch, 1), lambda i: (0, 0)),
        out_shape=jax.ShapeDtypeStruct((batch, 1), jnp.float32),
        scratch_shapes=[
            pltpu.VMEM((batch, 1), jnp.float32),
            pltpu.VMEM((batch, 1), jnp.float32),
        ],
    )(v, W, b2)


def _logsoftmax_out_tc(v, W, b2, lse, vocab):
    batch, k = v.shape
    nv = pl.cdiv(vocab, _VB)

    def body(v_ref, w_ref, b_ref, lse_ref, o_ref):
        logits = lax.dot_general(v_ref[:], w_ref[:], (((1,), (1,)), ((), ())),
                                 preferred_element_type=jnp.float32) + b_ref[:]
        o_ref[:] = logits - lse_ref[:]

    return pl.pallas_call(
        body,
        grid=(nv,),
        in_specs=[
            pl.BlockSpec((batch, k), lambda i: (0, 0)),
            pl.BlockSpec((_VB, k), lambda i: (i, 0)),
            pl.BlockSpec((1, _VB), lambda i: (0, i)),
            pl.BlockSpec((batch, 1), lambda i: (0, 0)),
        ],
        out_specs=pl.BlockSpec((batch, _VB), lambda i: (0, i)),
        out_shape=jax.ShapeDtypeStruct((batch, vocab), jnp.float32),
    )(v, W, b2, lse)


def kernel(x, emb, W, b):
    batch, win = x.shape
    vocab, edim = W.shape
    idx = x.reshape(-1)
    v = _gather_mean_sc(idx, emb, batch, win)
    b2 = b.reshape(1, vocab)
    lse = _lse_tc(v, W, b2, vocab)
    return _logsoftmax_out_tc(v, W, b2, lse, vocab)


# speedup vs baseline: 1.0128x; 1.0128x over previous
import functools

import jax
import jax.numpy as jnp
from jax import lax
from jax.experimental import pallas as pl
from jax.experimental.pallas import tpu as pltpu
from jax.experimental.pallas import tpu_sc as plsc

_VB = 2048
_IDX_CHUNK = 128


def _gather_mean_sc(x, emb, batch, win):
    vocab, edim = emb.shape
    rows128 = vocab * edim // 128
    per_row = 128 // edim
    emb128 = emb.reshape(rows128, 128)
    xf = x.reshape(-1)
    idx_main = xf // per_row
    lane_idx = ((xf % per_row)[:, None] * edim
                + jnp.arange(edim, dtype=jnp.int32)[None, :])

    info = plsc.get_sparse_core_info()
    nc, ns = info.num_cores, info.num_subcores
    nw = nc * ns
    b_per_w = batch // nw
    rows_per_w = b_per_w * win
    n_chunks = rows_per_w // _IDX_CHUNK
    mesh = plsc.VectorSubcoreMesh(core_axis_name="c", subcore_axis_name="s")

    ch_b = 8
    ch_r = ch_b * win
    n_ch = b_per_w // ch_b

    def body(idx_hbm, lane_hbm, emb_hbm, out_hbm, idx_v, lane_v, rows0,
             rows1, acc_v, sem0, sem1):
        wid = lax.axis_index("s") * nc + lax.axis_index("c")
        base = wid * rows_per_w
        pltpu.sync_copy(idx_hbm.at[pl.ds(base, rows_per_w)], idx_v)
        pltpu.sync_copy(lane_hbm.at[pl.ds(base, rows_per_w)], lane_v)
        rows = (rows0, rows1)
        sems = (sem0, sem1)

        def fire(c):
            cbase, off, hs = c * ch_r, 0, []
            while off < ch_r:
                ln = min(_IDX_CHUNK, ch_r - off)
                hs.append(pltpu.async_copy(
                    emb_hbm.at[idx_v.at[pl.ds(cbase + off, ln)]],
                    rows[c % 2].at[pl.ds(off, ln)],
                    sems[c % 2]))
                off += ln
            return hs

        pending = {0: fire(0)}
        for c in range(n_ch):
            for h in pending.pop(c):
                h.wait()
            if c + 1 < n_ch:
                pending[c + 1] = fire(c + 1)
            rv = rows[c % 2]

            def pool_one(i, carry, c=c, rv=rv):
                lr0 = i * win
                gr0 = c * ch_r + lr0
                acc = plsc.load_gather(
                    rv, [jnp.full((16,), lr0, jnp.int32), lane_v[gr0, :]])
                for j in range(1, win):
                    acc = acc + plsc.load_gather(
                        rv, [jnp.full((16,), lr0 + j, jnp.int32),
                             lane_v[gr0 + j, :]])
                acc_v[c * ch_b + i, :] = acc * (1.0 / win)
                return carry

            lax.fori_loop(0, ch_b, pool_one, 0)
        pltpu.sync_copy(acc_v, out_hbm.at[pl.ds(wid * b_per_w, b_per_w)])

    kfn = pl.kernel(
        body,
        mesh=mesh,
        compiler_params=pltpu.CompilerParams(needs_layout_passes=False),
        out_type=jax.ShapeDtypeStruct((batch, edim), jnp.float32),
        scratch_types=[
            pltpu.VMEM((rows_per_w,), jnp.int32),
            pltpu.VMEM((rows_per_w, edim), jnp.int32),
            pltpu.VMEM((ch_r, 128), jnp.float32),
            pltpu.VMEM((ch_r, 128), jnp.float32),
            pltpu.VMEM((b_per_w, edim), jnp.float32),
            pltpu.SemaphoreType.DMA,
            pltpu.SemaphoreType.DMA,
        ],
    )
    return kfn(idx_main, lane_idx, emb128)


def _lse_tc(v, W, b2, vocab):
    batch, k = v.shape
    nv = pl.cdiv(vocab, _VB)

    def body(v_ref, w_ref, b_ref, lse_ref, m_s, s_s):
        iv = pl.program_id(0)
        logits = lax.dot_general(v_ref[:], w_ref[:], (((1,), (1,)), ((), ())),
                                 preferred_element_type=jnp.float32) + b_ref[:]
        col = iv * _VB + lax.broadcasted_iota(jnp.int32, logits.shape, 1)
        logits = jnp.where(col < vocab, logits, -1e30)
        tmax = jnp.max(logits, axis=1, keepdims=True)

        @pl.when(iv == 0)
        def _():
            m_s[:] = tmax
            s_s[:] = jnp.sum(jnp.exp(logits - tmax), axis=1, keepdims=True)

        @pl.when(iv != 0)
        def _():
            m_old = m_s[:]
            m_new = jnp.maximum(m_old, tmax)
            s_s[:] = (s_s[:] * jnp.exp(m_old - m_new)
                      + jnp.sum(jnp.exp(logits - m_new), axis=1,
                                keepdims=True))
            m_s[:] = m_new

        @pl.when(iv == nv - 1)
        def _():
            lse_ref[:] = m_s[:] + jnp.log(s_s[:])

    return pl.pallas_call(
        body,
        grid=(nv,),
        in_specs=[
            pl.BlockSpec((batch, k), lambda i: (0, 0)),
            pl.BlockSpec((_VB, k), lambda i: (i, 0)),
            pl.BlockSpec((1, _VB), lambda i: (0, i)),
        ],
        out_specs=pl.BlockSpec((batch, 1), lambda i: (0, 0)),
        out_shape=jax.ShapeDtypeStruct((batch, 1), jnp.float32),
        scratch_shapes=[
            pltpu.VMEM((batch, 1), jnp.float32),
            pltpu.VMEM((batch, 1), jnp.float32),
        ],
    )(v, W, b2)


def _logsoftmax_out_tc(v, W, b2, lse, vocab):
    batch, k = v.shape
    nv = pl.cdiv(vocab, _VB)

    def body(v_ref, w_ref, b_ref, lse_ref, o_ref):
        logits = lax.dot_general(v_ref[:], w_ref[:], (((1,), (1,)), ((), ())),
                                 preferred_element_type=jnp.float32) + b_ref[:]
        o_ref[:] = logits - lse_ref[:]

    return pl.pallas_call(
        body,
        grid=(nv,),
        in_specs=[
            pl.BlockSpec((batch, k), lambda i: (0, 0)),
            pl.BlockSpec((_VB, k), lambda i: (i, 0)),
            pl.BlockSpec((1, _VB), lambda i: (0, i)),
            pl.BlockSpec((batch, 1), lambda i: (0, 0)),
        ],
        out_specs=pl.BlockSpec((batch, _VB), lambda i: (0, i)),
        out_shape=jax.ShapeDtypeStruct((batch, vocab), jnp.float32),
    )(v, W, b2, lse)


def kernel(x, emb, W, b):
    batch, win = x.shape
    vocab, edim = W.shape
    v = _gather_mean_sc(x, emb, batch, win)
    b2 = b.reshape(1, vocab)
    lse = _lse_tc(v, W, b2, vocab)
    return _logsoftmax_out_tc(v, W, b2, lse, vocab)


# speedup vs baseline: 2.1004x; 2.0738x over previous
import functools

import jax
import jax.numpy as jnp
from jax import lax
from jax.experimental import pallas as pl
from jax.experimental.pallas import tpu as pltpu
from jax.experimental.pallas import tpu_sc as plsc

_VB = 2048
_IDX_CHUNK = 128


def _gather_mean_sc(x, emb, batch, win):
    vocab, edim = emb.shape
    rows128 = vocab * edim // 128
    per_row = 128 // edim
    emb128 = emb.reshape(rows128, 128)
    xf = x.reshape(-1)
    idx_main = xf // per_row
    lane_idx = ((xf % per_row)[:, None] * edim
                + jnp.arange(edim, dtype=jnp.int32)[None, :])

    info = plsc.get_sparse_core_info()
    nc, ns = info.num_cores, info.num_subcores
    nw = nc * ns
    b_per_w = batch // nw
    rows_per_w = b_per_w * win
    n_chunks = rows_per_w // _IDX_CHUNK
    mesh = plsc.VectorSubcoreMesh(core_axis_name="c", subcore_axis_name="s")

    ch_b = 8
    ch_r = ch_b * win
    n_ch = b_per_w // ch_b

    def body(idx_hbm, lane_hbm, emb_hbm, out_hbm, idx_v, lane_v, rows0,
             rows1, acc_v, sem0, sem1):
        wid = lax.axis_index("s") * nc + lax.axis_index("c")
        base = wid * rows_per_w
        pltpu.sync_copy(idx_hbm.at[pl.ds(base, rows_per_w)], idx_v)
        pltpu.sync_copy(lane_hbm.at[pl.ds(base, rows_per_w)], lane_v)
        rows = (rows0, rows1)
        sems = (sem0, sem1)

        def fire(c):
            cbase, off, hs = c * ch_r, 0, []
            while off < ch_r:
                ln = min(_IDX_CHUNK, ch_r - off)
                hs.append(pltpu.async_copy(
                    emb_hbm.at[idx_v.at[pl.ds(cbase + off, ln)]],
                    rows[c % 2].at[pl.ds(off, ln)],
                    sems[c % 2]))
                off += ln
            return hs

        pending = {0: fire(0)}
        for c in range(n_ch):
            for h in pending.pop(c):
                h.wait()
            if c + 1 < n_ch:
                pending[c + 1] = fire(c + 1)
            rv = rows[c % 2]

            def pool_one(i, carry, c=c, rv=rv):
                lr0 = i * win
                gr0 = c * ch_r + lr0
                acc = plsc.load_gather(
                    rv, [jnp.full((16,), lr0, jnp.int32), lane_v[gr0, :]])
                for j in range(1, win):
                    acc = acc + plsc.load_gather(
                        rv, [jnp.full((16,), lr0 + j, jnp.int32),
                             lane_v[gr0 + j, :]])
                acc_v[c * ch_b + i, :] = acc * (1.0 / win)
                return carry

            lax.fori_loop(0, ch_b, pool_one, 0)
        pltpu.sync_copy(acc_v, out_hbm.at[pl.ds(wid * b_per_w, b_per_w)])

    kfn = pl.kernel(
        body,
        mesh=mesh,
        compiler_params=pltpu.CompilerParams(needs_layout_passes=False),
        out_type=jax.ShapeDtypeStruct((batch, edim), jnp.float32),
        scratch_types=[
            pltpu.VMEM((rows_per_w,), jnp.int32),
            pltpu.VMEM((rows_per_w, edim), jnp.int32),
            pltpu.VMEM((ch_r, 128), jnp.float32),
            pltpu.VMEM((ch_r, 128), jnp.float32),
            pltpu.VMEM((b_per_w, edim), jnp.float32),
            pltpu.SemaphoreType.DMA,
            pltpu.SemaphoreType.DMA,
        ],
    )
    return kfn(idx_main, lane_idx, emb128)


def _lse_tc(vb, wbt, vocab):
    batch, k = vb.shape
    nv = pl.cdiv(vocab, _VB)

    def body(wbt_ref, vb_ref, lse_ref, m_s, s_s):
        iv = pl.program_id(0)
        logits = lax.dot_general(wbt_ref[:], vb_ref[:], (((0,), (1,)), ((), ())),
                                 preferred_element_type=jnp.float32)
        row = iv * _VB + lax.broadcasted_iota(jnp.int32, logits.shape, 0)
        logits = jnp.where(row < vocab, logits, -1e30)
        tmax = jnp.max(logits, axis=0, keepdims=True)

        @pl.when(iv == 0)
        def _():
            m_s[:] = tmax
            s_s[:] = jnp.sum(jnp.exp(logits - tmax), axis=0, keepdims=True)

        @pl.when(iv != 0)
        def _():
            m_old = m_s[:]
            m_new = jnp.maximum(m_old, tmax)
            s_s[:] = (s_s[:] * jnp.exp(m_old - m_new)
                      + jnp.sum(jnp.exp(logits - m_new), axis=0,
                                keepdims=True))
            m_s[:] = m_new

        @pl.when(iv == nv - 1)
        def _():
            lse_ref[:] = m_s[:] + jnp.log(s_s[:])

    return pl.pallas_call(
        body,
        grid=(nv,),
        in_specs=[
            pl.BlockSpec((k, _VB), lambda i: (0, i)),
            pl.BlockSpec((batch, k), lambda i: (0, 0)),
        ],
        out_specs=pl.BlockSpec((1, batch), lambda i: (0, 0)),
        out_shape=jax.ShapeDtypeStruct((1, batch), jnp.float32),
        scratch_shapes=[
            pltpu.VMEM((1, batch), jnp.float32),
            pltpu.VMEM((1, batch), jnp.float32),
        ],
    )(wbt, vb)


def _logsoftmax_out_tc(vb, wbt, lse, vocab):
    batch, k = vb.shape
    nv = pl.cdiv(vocab, _VB)

    def body(wbt_ref, vb_ref, lse_ref, o_ref):
        logits = lax.dot_general(wbt_ref[:], vb_ref[:], (((0,), (1,)), ((), ())),
                                 preferred_element_type=jnp.float32)
        o_ref[:] = logits - lse_ref[:]

    return pl.pallas_call(
        body,
        grid=(nv,),
        in_specs=[
            pl.BlockSpec((k, _VB), lambda i: (0, i)),
            pl.BlockSpec((batch, k), lambda i: (0, 0)),
            pl.BlockSpec((1, batch), lambda i: (0, 0)),
        ],
        out_specs=pl.BlockSpec((_VB, batch), lambda i: (i, 0)),
        out_shape=jax.ShapeDtypeStruct((vocab, batch), jnp.float32),
    )(wbt, vb, lse)


def kernel(x, emb, W, b):
    batch, win = x.shape
    vocab, edim = W.shape
    v = _gather_mean_sc(x, emb, batch, win)
    wbt = jnp.concatenate([W.T, b[None, :]], axis=0)
    vb = jnp.concatenate([v, jnp.ones((batch, 1), jnp.float32)], axis=1)
    lse = _lse_tc(vb, wbt, vocab)
    out_t = _logsoftmax_out_tc(vb, wbt, lse, vocab)
    return out_t.T


# speedup vs baseline: 2.1060x; 1.0027x over previous
import functools

import jax
import jax.numpy as jnp
from jax import lax
from jax.experimental import pallas as pl
from jax.experimental.pallas import tpu as pltpu
from jax.experimental.pallas import tpu_sc as plsc

_VB = 2048
_IDX_CHUNK = 128


def _gather_mean_sc(x, emb, batch, win):
    vocab, edim = emb.shape
    rows128 = vocab * edim // 128
    per_row = 128 // edim
    emb128 = emb.reshape(rows128, 128)
    xf = x.reshape(-1)
    idx_main = xf // per_row
    lane_idx = ((xf % per_row)[:, None] * edim
                + jnp.arange(edim, dtype=jnp.int32)[None, :])

    info = plsc.get_sparse_core_info()
    nc, ns = info.num_cores, info.num_subcores
    nw = nc * ns
    b_per_w = batch // nw
    rows_per_w = b_per_w * win
    n_chunks = rows_per_w // _IDX_CHUNK
    mesh = plsc.VectorSubcoreMesh(core_axis_name="c", subcore_axis_name="s")

    ch_b = 8
    ch_r = ch_b * win
    n_ch = b_per_w // ch_b

    def body(idx_hbm, lane_hbm, emb_hbm, out_hbm, idx_v, lane_v, rows0,
             rows1, acc_v, sem0, sem1):
        wid = lax.axis_index("s") * nc + lax.axis_index("c")
        base = wid * rows_per_w
        pltpu.sync_copy(idx_hbm.at[pl.ds(base, rows_per_w)], idx_v)
        pltpu.sync_copy(lane_hbm.at[pl.ds(base, rows_per_w)], lane_v)
        rows = (rows0, rows1)
        sems = (sem0, sem1)

        def fire(c):
            cbase, off, hs = c * ch_r, 0, []
            while off < ch_r:
                ln = min(_IDX_CHUNK, ch_r - off)
                hs.append(pltpu.async_copy(
                    emb_hbm.at[idx_v.at[pl.ds(cbase + off, ln)]],
                    rows[c % 2].at[pl.ds(off, ln)],
                    sems[c % 2]))
                off += ln
            return hs

        pending = {0: fire(0)}
        for c in range(n_ch):
            for h in pending.pop(c):
                h.wait()
            if c + 1 < n_ch:
                pending[c + 1] = fire(c + 1)
            rv = rows[c % 2]

            def pool_one(i, carry, c=c, rv=rv):
                lr0 = i * win
                gr0 = c * ch_r + lr0
                acc = plsc.load_gather(
                    rv, [jnp.full((16,), lr0, jnp.int32), lane_v[gr0, :]])
                for j in range(1, win):
                    acc = acc + plsc.load_gather(
                        rv, [jnp.full((16,), lr0 + j, jnp.int32),
                             lane_v[gr0 + j, :]])
                acc_v[c * ch_b + i, :] = acc * (1.0 / win)
                return carry

            lax.fori_loop(0, ch_b, pool_one, 0)
        pltpu.sync_copy(acc_v, out_hbm.at[pl.ds(wid * b_per_w, b_per_w)])

    kfn = pl.kernel(
        body,
        mesh=mesh,
        compiler_params=pltpu.CompilerParams(needs_layout_passes=False),
        out_type=jax.ShapeDtypeStruct((batch, edim), jnp.float32),
        scratch_types=[
            pltpu.VMEM((rows_per_w,), jnp.int32),
            pltpu.VMEM((rows_per_w, edim), jnp.int32),
            pltpu.VMEM((ch_r, 128), jnp.float32),
            pltpu.VMEM((ch_r, 128), jnp.float32),
            pltpu.VMEM((b_per_w, edim), jnp.float32),
            pltpu.SemaphoreType.DMA,
            pltpu.SemaphoreType.DMA,
        ],
    )
    return kfn(idx_main, lane_idx, emb128)


def _lse_tc(vb, wbt, vocab):
    batch, k = vb.shape
    nv = wbt.shape[1] // _VB

    def body(wbt_ref, vb_ref, lse_ref, m_s, s_s):
        iv = pl.program_id(0)
        logits = lax.dot_general(wbt_ref[:], vb_ref[:], (((0,), (1,)), ((), ())),
                                 preferred_element_type=jnp.float32)
        tmax = jnp.max(logits, axis=0, keepdims=True)

        @pl.when(iv == 0)
        def _():
            m_s[:] = tmax
            s_s[:] = jnp.sum(jnp.exp(logits - tmax), axis=0, keepdims=True)

        @pl.when(iv != 0)
        def _():
            m_old = m_s[:]
            m_new = jnp.maximum(m_old, tmax)
            s_s[:] = (s_s[:] * jnp.exp(m_old - m_new)
                      + jnp.sum(jnp.exp(logits - m_new), axis=0,
                                keepdims=True))
            m_s[:] = m_new

        @pl.when(iv == nv - 1)
        def _():
            lse_ref[:] = m_s[:] + jnp.log(s_s[:])

    return pl.pallas_call(
        body,
        grid=(nv,),
        in_specs=[
            pl.BlockSpec((k, _VB), lambda i: (0, i)),
            pl.BlockSpec((batch, k), lambda i: (0, 0)),
        ],
        out_specs=pl.BlockSpec((1, batch), lambda i: (0, 0)),
        out_shape=jax.ShapeDtypeStruct((1, batch), jnp.float32),
        scratch_shapes=[
            pltpu.VMEM((1, batch), jnp.float32),
            pltpu.VMEM((1, batch), jnp.float32),
        ],
    )(wbt, vb)


def _logsoftmax_out_tc(vb, wbt, lse, vocab):
    batch, k = vb.shape
    nv = pl.cdiv(vocab, _VB)

    def body(wbt_ref, vb_ref, lse_ref, o_ref):
        logits = lax.dot_general(wbt_ref[:], vb_ref[:], (((0,), (1,)), ((), ())),
                                 preferred_element_type=jnp.float32)
        o_ref[:] = logits - lse_ref[:]

    return pl.pallas_call(
        body,
        grid=(nv,),
        in_specs=[
            pl.BlockSpec((k, _VB), lambda i: (0, i)),
            pl.BlockSpec((batch, k), lambda i: (0, 0)),
            pl.BlockSpec((1, batch), lambda i: (0, 0)),
        ],
        out_specs=pl.BlockSpec((_VB, batch), lambda i: (i, 0)),
        out_shape=jax.ShapeDtypeStruct((vocab, batch), jnp.float32),
    )(wbt, vb, lse)


def kernel(x, emb, W, b):
    batch, win = x.shape
    vocab, edim = W.shape
    v = _gather_mean_sc(x, emb, batch, win)
    extra = _VB * pl.cdiv(vocab, _VB) - vocab
    wbt = jnp.concatenate(
        [jnp.concatenate([W.T, b[None, :]], axis=0),
         jnp.concatenate([jnp.zeros((edim, extra), jnp.float32),
                          jnp.full((1, extra), -1e30, jnp.float32)], axis=0)],
        axis=1)
    vb = jnp.concatenate([v, jnp.ones((batch, 1), jnp.float32)], axis=1)
    lse = _lse_tc(vb, wbt, vocab)
    out_t = _logsoftmax_out_tc(vb, wbt, lse, vocab)
    return out_t.T
